# Initial kernel scaffold; baseline (speedup 1.0000x reference)
import jax
import jax.numpy as jnp
from jax.experimental import pallas as pl


def kernel(x, edge_index):
    raise NotImplementedError("write your pallas kernel here")


# speedup vs baseline: 3.5235x; 3.5235x over previous
import functools

import jax
import jax.numpy as jnp
from jax import lax
from jax.experimental import pallas as pl
from jax.experimental.pallas import tpu as pltpu
from jax.experimental.pallas import tpu_sc as plsc

N_NODES = 10000
D_FEAT = 128
N_EDGES = 320000

N_TILES = 16
EDGES_PER_TILE = N_EDGES // N_TILES
CHUNK = 128
N_FULL = EDGES_PER_TILE // CHUNK
TAIL = EDGES_PER_TILE - N_FULL * CHUNK
N_PAD = 10240
ROWS_PER_TILE = N_PAD // N_TILES
WCHUNK = 128
N_WCHUNKS = ROWS_PER_TILE // WCHUNK
LANES = 16

_mesh = plsc.VectorSubcoreMesh(
    core_axis_name="c", subcore_axis_name="s", num_cores=1
)


@functools.partial(
    pl.kernel,
    out_type=(
        jax.ShapeDtypeStruct((N_PAD, D_FEAT), jnp.float32),
        jax.ShapeDtypeStruct((N_PAD, D_FEAT), jnp.float32),
        jax.ShapeDtypeStruct((N_PAD, D_FEAT), jnp.float32),
    ),
    mesh=_mesh,
    scratch_types=[
        pltpu.VMEM_SHARED((N_PAD, D_FEAT), jnp.float32),
        pltpu.VMEM((CHUNK,), jnp.int32),
        pltpu.VMEM((CHUNK,), jnp.int32),
        pltpu.VMEM((CHUNK, D_FEAT), jnp.float32),
        pltpu.VMEM((TAIL,), jnp.int32),
        pltpu.VMEM((TAIL,), jnp.int32),
        pltpu.VMEM((TAIL, D_FEAT), jnp.float32),
        pltpu.VMEM((WCHUNK, D_FEAT), jnp.float32),
    ],
)
def _conv3(x, src, dst, h1, h2, out,
           acc, idx_s, idx_d, rows, idx_s_t, idx_d_t, rows_t, wbuf):
    wid = lax.axis_index("s")
    ebase = wid * EDGES_PER_TILE
    rbase = wid * ROWS_PER_TILE

    zeros = jnp.zeros((LANES,), jnp.float32)

    def layer(src_buf, dst_buf, relu):
        @pl.loop(0, WCHUNK)
        def _(r):
            for c in range(D_FEAT // LANES):
                wbuf[r, pl.ds(c * LANES, LANES)] = zeros

        for k in range(N_WCHUNKS):
            pltpu.sync_copy(wbuf, acc.at[pl.ds(rbase + k * WCHUNK, WCHUNK)])
        plsc.subcore_barrier()

        @pl.loop(0, N_FULL)
        def _(i):
            off = ebase + i * CHUNK
            pltpu.sync_copy(src.at[pl.ds(off, CHUNK)], idx_s)
            pltpu.sync_copy(dst.at[pl.ds(off, CHUNK)], idx_d)
            pltpu.sync_copy(src_buf.at[idx_s], rows)
            pltpu.sync_copy(rows, acc.at[idx_d], add=True)

        if TAIL:
            off = ebase + N_FULL * CHUNK
            pltpu.sync_copy(src.at[pl.ds(off, TAIL)], idx_s_t)
            pltpu.sync_copy(dst.at[pl.ds(off, TAIL)], idx_d_t)
            pltpu.sync_copy(src_buf.at[idx_s_t], rows_t)
            pltpu.sync_copy(rows_t, acc.at[idx_d_t], add=True)
        plsc.subcore_barrier()

        for k in range(N_WCHUNKS):
            r0 = rbase + k * WCHUNK
            pltpu.sync_copy(acc.at[pl.ds(r0, WCHUNK)], wbuf)
            if relu:
                @pl.loop(0, WCHUNK)
                def _(r):
                    for c in range(D_FEAT // LANES):
                        v = wbuf[r, pl.ds(c * LANES, LANES)]
                        wbuf[r, pl.ds(c * LANES, LANES)] = jnp.maximum(v, 0.0)
            pltpu.sync_copy(wbuf, dst_buf.at[pl.ds(r0, WCHUNK)])
        plsc.subcore_barrier()

    layer(x, h1, True)
    layer(h1, h2, False)
    layer(h2, out, False)


def kernel(x, edge_index):
    src = edge_index[0].astype(jnp.int32)
    dst = edge_index[1].astype(jnp.int32)
    _, _, out = _conv3(x, src, dst)
    return out[:N_NODES]


# speedup vs baseline: 4.4442x; 1.2613x over previous
import functools

import jax
import jax.numpy as jnp
from jax import lax
from jax.experimental import pallas as pl
from jax.experimental.pallas import tpu as pltpu
from jax.experimental.pallas import tpu_sc as plsc

N_NODES = 10000
D_FEAT = 128
HALF = D_FEAT // 2
N_EDGES = 320000

N_TILES = 16
EDGES_PER_TILE = N_EDGES // N_TILES
CHUNK = 128
N_FULL = EDGES_PER_TILE // CHUNK
TAIL = EDGES_PER_TILE - N_FULL * CHUNK
N_PAD = 10240
ROWS_PER_TILE = N_PAD // N_TILES
WCHUNK = 128
N_WCHUNKS = ROWS_PER_TILE // WCHUNK
LANES = 16

_mesh = plsc.VectorSubcoreMesh(
    core_axis_name="c", subcore_axis_name="s", num_cores=2
)

_half = jax.ShapeDtypeStruct((N_PAD, HALF), jnp.float32)


@functools.partial(
    pl.kernel,
    out_type=(_half,) * 6,
    mesh=_mesh,
    compiler_params=pltpu.CompilerParams(use_tc_tiling_on_sc=False),
    scratch_types=[
        pltpu.VMEM_SHARED((N_PAD, HALF), jnp.float32),
        pltpu.VMEM((CHUNK,), jnp.int32),
        pltpu.VMEM((CHUNK,), jnp.int32),
        pltpu.VMEM((CHUNK, HALF), jnp.float32),
        pltpu.VMEM((TAIL,), jnp.int32),
        pltpu.VMEM((TAIL,), jnp.int32),
        pltpu.VMEM((TAIL, HALF), jnp.float32),
        pltpu.VMEM((WCHUNK, HALF), jnp.float32),
    ],
)
def _conv3(x_lo, x_hi, src, dst,
           h1_lo, h1_hi, h2_lo, h2_hi, o_lo, o_hi,
           acc, idx_s, idx_d, rows, idx_s_t, idx_d_t, rows_t, wbuf):
    cid = lax.axis_index("c")
    wid = lax.axis_index("s")
    ebase = wid * EDGES_PER_TILE
    rbase = wid * ROWS_PER_TILE

    zeros = jnp.zeros((LANES,), jnp.float32)

    def layer(src_buf, dst_buf, relu):
        @pl.loop(0, WCHUNK)
        def _(r):
            for c in range(HALF // LANES):
                wbuf[r, pl.ds(c * LANES, LANES)] = zeros

        for k in range(N_WCHUNKS):
            pltpu.sync_copy(wbuf, acc.at[pl.ds(rbase + k * WCHUNK, WCHUNK)])
        plsc.subcore_barrier()

        @pl.loop(0, N_FULL)
        def _(i):
            off = ebase + i * CHUNK
            pltpu.sync_copy(src.at[pl.ds(off, CHUNK)], idx_s)
            pltpu.sync_copy(dst.at[pl.ds(off, CHUNK)], idx_d)
            pltpu.sync_copy(src_buf.at[idx_s], rows)
            pltpu.sync_copy(rows, acc.at[idx_d], add=True)

        if TAIL:
            off = ebase + N_FULL * CHUNK
            pltpu.sync_copy(src.at[pl.ds(off, TAIL)], idx_s_t)
            pltpu.sync_copy(dst.at[pl.ds(off, TAIL)], idx_d_t)
            pltpu.sync_copy(src_buf.at[idx_s_t], rows_t)
            pltpu.sync_copy(rows_t, acc.at[idx_d_t], add=True)
        plsc.subcore_barrier()

        for k in range(N_WCHUNKS):
            r0 = rbase + k * WCHUNK
            pltpu.sync_copy(acc.at[pl.ds(r0, WCHUNK)], wbuf)
            if relu:
                @pl.loop(0, WCHUNK)
                def _(r):
                    for c in range(HALF // LANES):
                        v = wbuf[r, pl.ds(c * LANES, LANES)]
                        wbuf[r, pl.ds(c * LANES, LANES)] = jnp.maximum(v, 0.0)
            pltpu.sync_copy(wbuf, dst_buf.at[pl.ds(r0, WCHUNK)])
        plsc.subcore_barrier()

    @pl.when(cid == 0)
    def _():
        layer(x_lo, h1_lo, True)
        layer(h1_lo, h2_lo, False)
        layer(h2_lo, o_lo, False)

    @pl.when(cid == 1)
    def _():
        layer(x_hi, h1_hi, True)
        layer(h1_hi, h2_hi, False)
        layer(h2_hi, o_hi, False)


def kernel(x, edge_index):
    src = edge_index[0].astype(jnp.int32)
    dst = edge_index[1].astype(jnp.int32)
    x_lo = x[:, :HALF]
    x_hi = x[:, HALF:]
    *_, o_lo, o_hi = _conv3(x_lo, x_hi, src, dst)
    return jnp.concatenate([o_lo[:N_NODES], o_hi[:N_NODES]], axis=1)


# speedup vs baseline: 7.0712x; 1.5911x over previous
import functools

import jax
import jax.numpy as jnp
from jax import lax
from jax.experimental import pallas as pl
from jax.experimental.pallas import tpu as pltpu
from jax.experimental.pallas import tpu_sc as plsc

N_NODES = 10000
D_FEAT = 128
HALF = D_FEAT // 2
N_EDGES = 320000

N_TILES = 16
EDGES_PER_TILE = N_EDGES // N_TILES
CHUNK = 128
N_FULL = EDGES_PER_TILE // CHUNK
TAIL = EDGES_PER_TILE - N_FULL * CHUNK
N_PAD = 10240
ROWS_PER_TILE = N_PAD // N_TILES
WCHUNK = 128
N_WCHUNKS = ROWS_PER_TILE // WCHUNK
LANES = 16

_mesh = plsc.VectorSubcoreMesh(
    core_axis_name="c", subcore_axis_name="s", num_cores=2
)

_half = jax.ShapeDtypeStruct((N_PAD, HALF), jnp.float32)


@functools.partial(
    pl.kernel,
    out_type=(_half,) * 6,
    mesh=_mesh,
    compiler_params=pltpu.CompilerParams(use_tc_tiling_on_sc=False),
    scratch_types=[
        pltpu.VMEM_SHARED((N_PAD, HALF), jnp.float32),
        pltpu.VMEM((CHUNK,), jnp.int32),
        pltpu.VMEM((CHUNK,), jnp.int32),
        pltpu.VMEM((CHUNK, HALF), jnp.float32),
        pltpu.VMEM((CHUNK,), jnp.int32),
        pltpu.VMEM((CHUNK,), jnp.int32),
        pltpu.VMEM((CHUNK, HALF), jnp.float32),
        pltpu.VMEM((TAIL,), jnp.int32),
        pltpu.VMEM((TAIL,), jnp.int32),
        pltpu.VMEM((TAIL, HALF), jnp.float32),
        pltpu.VMEM((WCHUNK, HALF), jnp.float32),
        pltpu.SemaphoreType.DMA,
        pltpu.SemaphoreType.DMA,
    ],
)
def _conv3(x_lo, x_hi, src, dst,
           h1_lo, h1_hi, h2_lo, h2_hi, o_lo, o_hi,
           acc, idx_s0, idx_d0, rows0, idx_s1, idx_d1, rows1,
           idx_s_t, idx_d_t, rows_t, wbuf, sem0, sem1):
    cid = lax.axis_index("c")
    wid = lax.axis_index("s")
    ebase = wid * EDGES_PER_TILE
    rbase = wid * ROWS_PER_TILE

    zeros = jnp.zeros((LANES,), jnp.float32)

    def layer(src_buf, dst_buf, relu):
        @pl.loop(0, WCHUNK)
        def _(r):
            for c in range(HALF // LANES):
                wbuf[r, pl.ds(c * LANES, LANES)] = zeros

        for k in range(N_WCHUNKS):
            pltpu.sync_copy(wbuf, acc.at[pl.ds(rbase + k * WCHUNK, WCHUNK)])
        plsc.subcore_barrier()

        bufs = ((idx_s0, idx_d0, rows0, sem0), (idx_s1, idx_d1, rows1, sem1))

        def stage_and_launch(b, i):
            s_ref, d_ref, r_ref, sem = bufs[b]
            off = ebase + i * CHUNK
            pltpu.sync_copy(src.at[pl.ds(off, CHUNK)], s_ref)
            pltpu.sync_copy(dst.at[pl.ds(off, CHUNK)], d_ref)
            pltpu.async_copy(src_buf.at[s_ref], r_ref, sem)

        def finish(b):
            s_ref, d_ref, r_ref, sem = bufs[b]
            pltpu.make_async_copy(src_buf.at[s_ref], r_ref, sem).wait()
            pltpu.sync_copy(r_ref, acc.at[d_ref], add=True)

        stage_and_launch(0, 0)

        @pl.loop(0, N_FULL // 2 - 1)
        def _(j):
            i0 = 2 * j
            stage_and_launch(1, i0 + 1)
            finish(0)
            stage_and_launch(0, i0 + 2)
            finish(1)

        stage_and_launch(1, N_FULL - 1)
        finish(0)
        finish(1)

        if TAIL:
            off = ebase + N_FULL * CHUNK
            pltpu.sync_copy(src.at[pl.ds(off, TAIL)], idx_s_t)
            pltpu.sync_copy(dst.at[pl.ds(off, TAIL)], idx_d_t)
            pltpu.sync_copy(src_buf.at[idx_s_t], rows_t)
            pltpu.sync_copy(rows_t, acc.at[idx_d_t], add=True)
        plsc.subcore_barrier()

        for k in range(N_WCHUNKS):
            r0 = rbase + k * WCHUNK
            pltpu.sync_copy(acc.at[pl.ds(r0, WCHUNK)], wbuf)
            if relu:
                @pl.loop(0, WCHUNK)
                def _(r):
                    for c in range(HALF // LANES):
                        v = wbuf[r, pl.ds(c * LANES, LANES)]
                        wbuf[r, pl.ds(c * LANES, LANES)] = jnp.maximum(v, 0.0)
            pltpu.sync_copy(wbuf, dst_buf.at[pl.ds(r0, WCHUNK)])
        plsc.subcore_barrier()

    @pl.when(cid == 0)
    def _():
        layer(x_lo, h1_lo, True)
        layer(h1_lo, h2_lo, False)
        layer(h2_lo, o_lo, False)

    @pl.when(cid == 1)
    def _():
        layer(x_hi, h1_hi, True)
        layer(h1_hi, h2_hi, False)
        layer(h2_hi, o_hi, False)


def kernel(x, edge_index):
    src = edge_index[0].astype(jnp.int32)
    dst = edge_index[1].astype(jnp.int32)
    x_lo = x[:, :HALF]
    x_hi = x[:, HALF:]
    *_, o_lo, o_hi = _conv3(x_lo, x_hi, src, dst)
    return jnp.concatenate([o_lo[:N_NODES], o_hi[:N_NODES]], axis=1)


# speedup vs baseline: 12.2808x; 1.7367x over previous
import functools

import jax
import jax.numpy as jnp
from jax import lax
from jax.experimental import pallas as pl
from jax.experimental.pallas import tpu as pltpu
from jax.experimental.pallas import tpu_sc as plsc

N_NODES = 10000
D_FEAT = 128
HALF = D_FEAT // 2
N_EDGES = 320000

N_TILES = 16
CHUNK = 128
N_CHUNKS = N_EDGES // CHUNK
CHUNKS_PER_TILE = N_CHUNKS // N_TILES
EDGES_PER_TILE = CHUNKS_PER_TILE * CHUNK
BLK = 4
BLK_E = BLK * CHUNK
N_BLKS = CHUNKS_PER_TILE // BLK
EXTRA_TILES = N_CHUNKS - N_TILES * CHUNKS_PER_TILE
EXTRA_BASE = N_TILES * EDGES_PER_TILE
N_PAD = 10240
ROWS_PER_TILE = N_PAD // N_TILES
WCHUNK = 128
N_WCHUNKS = ROWS_PER_TILE // WCHUNK
LANES = 16

_mesh = plsc.VectorSubcoreMesh(
    core_axis_name="c", subcore_axis_name="s", num_cores=2
)

_half = jax.ShapeDtypeStruct((N_PAD, HALF), jnp.float32)


@functools.partial(
    pl.kernel,
    out_type=(_half,) * 6,
    mesh=_mesh,
    compiler_params=pltpu.CompilerParams(use_tc_tiling_on_sc=False),
    scratch_types=[
        pltpu.VMEM_SHARED((N_PAD, HALF), jnp.float32),
        pltpu.VMEM((BLK_E,), jnp.int32),
        pltpu.VMEM((BLK_E,), jnp.int32),
        pltpu.VMEM((BLK_E, HALF), jnp.float32),
        pltpu.VMEM((BLK_E,), jnp.int32),
        pltpu.VMEM((BLK_E,), jnp.int32),
        pltpu.VMEM((BLK_E, HALF), jnp.float32),
        pltpu.VMEM((WCHUNK, HALF), jnp.float32),
        pltpu.SemaphoreType.DMA,
        pltpu.SemaphoreType.DMA,
        pltpu.SemaphoreType.DMA,
        pltpu.SemaphoreType.DMA,
    ],
)
def _conv3(x_lo, x_hi, src, dst,
           h1_lo, h1_hi, h2_lo, h2_hi, o_lo, o_hi,
           acc, idx_s0, idx_d0, rows0, idx_s1, idx_d1, rows1,
           wbuf, gsem0, gsem1, ssem0, ssem1):
    cid = lax.axis_index("c")
    wid = lax.axis_index("s")
    ebase = wid * EDGES_PER_TILE
    rbase = wid * ROWS_PER_TILE

    zeros = jnp.zeros((LANES,), jnp.float32)

    def layer(src_buf, dst_buf, relu):
        @pl.loop(0, WCHUNK)
        def _(r):
            for c in range(HALF // LANES):
                wbuf[r, pl.ds(c * LANES, LANES)] = zeros

        for k in range(N_WCHUNKS):
            pltpu.sync_copy(wbuf, acc.at[pl.ds(rbase + k * WCHUNK, WCHUNK)])
        plsc.subcore_barrier()

        bufs = ((idx_s0, idx_d0, rows0, gsem0, ssem0),
                (idx_s1, idx_d1, rows1, gsem1, ssem1))

        def stage_launch(b, j):
            s_ref, d_ref, r_ref, gsem, _ = bufs[b]
            off = ebase + j * BLK_E
            pltpu.sync_copy(src.at[pl.ds(off, BLK_E)], s_ref)
            pltpu.sync_copy(dst.at[pl.ds(off, BLK_E)], d_ref)
            for k in range(BLK):
                sl = pl.ds(k * CHUNK, CHUNK)
                pltpu.async_copy(src_buf.at[s_ref.at[sl]], r_ref.at[sl], gsem)

        def finish(b):
            s_ref, d_ref, r_ref, gsem, ssem = bufs[b]
            for k in range(BLK):
                sl = pl.ds(k * CHUNK, CHUNK)
                pltpu.make_async_copy(
                    src_buf.at[s_ref.at[sl]], r_ref.at[sl], gsem).wait()
            descs = []
            for k in range(BLK):
                sl = pl.ds(k * CHUNK, CHUNK)
                descs.append(pltpu.async_copy(
                    r_ref.at[sl], acc.at[d_ref.at[sl]], ssem, add=True))
            for d in descs:
                d.wait()

        @pl.when(wid < EXTRA_TILES)
        def _():
            s_ref, d_ref, r_ref, gsem, _ = bufs[0]
            off = EXTRA_BASE + wid * CHUNK
            csl = pl.ds(0, CHUNK)
            pltpu.sync_copy(src.at[pl.ds(off, CHUNK)], s_ref.at[csl])
            pltpu.sync_copy(dst.at[pl.ds(off, CHUNK)], d_ref.at[csl])
            pltpu.sync_copy(src_buf.at[s_ref.at[csl]], r_ref.at[csl])
            pltpu.sync_copy(r_ref.at[csl], acc.at[d_ref.at[csl]], add=True)

        stage_launch(0, 0)

        @pl.loop(0, (N_BLKS - 1) // 2)
        def _(j):
            j0 = 2 * j
            stage_launch(1, j0 + 1)
            finish(0)
            stage_launch(0, j0 + 2)
            finish(1)

        finish(0)
        plsc.subcore_barrier()

        for k in range(N_WCHUNKS):
            r0 = rbase + k * WCHUNK
            pltpu.sync_copy(acc.at[pl.ds(r0, WCHUNK)], wbuf)
            if relu:
                @pl.loop(0, WCHUNK)
                def _(r):
                    for c in range(HALF // LANES):
                        v = wbuf[r, pl.ds(c * LANES, LANES)]
                        wbuf[r, pl.ds(c * LANES, LANES)] = jnp.maximum(v, 0.0)
            pltpu.sync_copy(wbuf, dst_buf.at[pl.ds(r0, WCHUNK)])
        plsc.subcore_barrier()

    @pl.when(cid == 0)
    def _():
        layer(x_lo, h1_lo, True)
        layer(h1_lo, h2_lo, False)
        layer(h2_lo, o_lo, False)

    @pl.when(cid == 1)
    def _():
        layer(x_hi, h1_hi, True)
        layer(h1_hi, h2_hi, False)
        layer(h2_hi, o_hi, False)


def kernel(x, edge_index):
    src = edge_index[0].astype(jnp.int32)
    dst = edge_index[1].astype(jnp.int32)
    x_lo = x[:, :HALF]
    x_hi = x[:, HALF:]
    *_, o_lo, o_hi = _conv3(x_lo, x_hi, src, dst)
    return jnp.concatenate([o_lo[:N_NODES], o_hi[:N_NODES]], axis=1)


# speedup vs baseline: 15.1692x; 1.2352x over previous
import functools

import jax
import jax.numpy as jnp
from jax import lax
from jax.experimental import pallas as pl
from jax.experimental.pallas import tpu as pltpu
from jax.experimental.pallas import tpu_sc as plsc

N_NODES = 10000
D_FEAT = 128
HALF = D_FEAT // 2
N_EDGES = 320000

N_TILES = 16
CHUNK = 128
N_CHUNKS = N_EDGES // CHUNK
CHUNKS_PER_TILE = N_CHUNKS // N_TILES
EDGES_PER_TILE = CHUNKS_PER_TILE * CHUNK
BLK = 4
BLK_E = BLK * CHUNK
N_BLKS = CHUNKS_PER_TILE // BLK
EXTRA_TILES = N_CHUNKS - N_TILES * CHUNKS_PER_TILE
EXTRA_BASE = N_TILES * EDGES_PER_TILE
N_PAD = 10240
ROWS_PER_TILE = N_PAD // N_TILES
WCHUNK = 128
N_WCHUNKS = ROWS_PER_TILE // WCHUNK
LANES = 16

_mesh = plsc.VectorSubcoreMesh(
    core_axis_name="c", subcore_axis_name="s", num_cores=2
)

_half = jax.ShapeDtypeStruct((N_PAD, HALF), jnp.float32)


@functools.partial(
    pl.kernel,
    out_type=(_half,) * 6,
    mesh=_mesh,
    compiler_params=pltpu.CompilerParams(use_tc_tiling_on_sc=False),
    scratch_types=[
        pltpu.VMEM_SHARED((N_PAD, HALF), jnp.float32),
        pltpu.VMEM((BLK_E, HALF), jnp.float32),
        pltpu.VMEM((BLK_E, HALF), jnp.float32),
        [pltpu.VMEM((BLK_E,), jnp.int32)] * 4,
        [pltpu.VMEM((BLK_E,), jnp.int32)] * 4,
        [pltpu.SemaphoreType.DMA] * 4,
        pltpu.VMEM((WCHUNK, HALF), jnp.float32),
        pltpu.SemaphoreType.DMA,
        pltpu.SemaphoreType.DMA,
        pltpu.SemaphoreType.DMA,
        pltpu.SemaphoreType.DMA,
    ],
)
def _conv3(x_lo, x_hi, src, dst,
           h1_lo, h1_hi, h2_lo, h2_hi, o_lo, o_hi,
           acc, rows0, rows1, idx_s, idx_d, isem,
           wbuf, gsem0, gsem1, ssem0, ssem1):
    cid = lax.axis_index("c")
    wid = lax.axis_index("s")
    ebase = wid * EDGES_PER_TILE
    rbase = wid * ROWS_PER_TILE

    zeros = jnp.zeros((LANES,), jnp.float32)

    def layer(src_buf, dst_buf, relu):
        @pl.loop(0, WCHUNK)
        def _(r):
            for c in range(HALF // LANES):
                wbuf[r, pl.ds(c * LANES, LANES)] = zeros

        for k in range(N_WCHUNKS):
            pltpu.sync_copy(wbuf, acc.at[pl.ds(rbase + k * WCHUNK, WCHUNK)])
        plsc.subcore_barrier()

        rbufs = ((rows0, gsem0, ssem0), (rows1, gsem1, ssem1))

        def fire_idx(s, blk):
            off = ebase + blk * BLK_E
            pltpu.async_copy(src.at[pl.ds(off, BLK_E)], idx_s[s], isem[s])
            pltpu.async_copy(dst.at[pl.ds(off, BLK_E)], idx_d[s], isem[s])

        def wait_idx(s):
            pltpu.make_async_copy(src.at[pl.ds(0, BLK_E)], idx_s[s], isem[s]).wait()
            pltpu.make_async_copy(dst.at[pl.ds(0, BLK_E)], idx_d[s], isem[s]).wait()

        def launch_g(b, s):
            r_ref, gsem, _ = rbufs[b]
            wait_idx(s)
            for k in range(BLK):
                sl = pl.ds(k * CHUNK, CHUNK)
                pltpu.async_copy(src_buf.at[idx_s[s].at[sl]], r_ref.at[sl], gsem)

        def finish(b, s):
            r_ref, gsem, ssem = rbufs[b]
            descs = []
            for k in range(BLK):
                sl = pl.ds(k * CHUNK, CHUNK)
                pltpu.make_async_copy(
                    src_buf.at[idx_s[s].at[sl]], r_ref.at[sl], gsem).wait()
                descs.append(pltpu.async_copy(
                    r_ref.at[sl], acc.at[idx_d[s].at[sl]], ssem, add=True))
            for d in descs:
                d.wait()

        @pl.when(wid < EXTRA_TILES)
        def _():
            off = EXTRA_BASE + wid * CHUNK
            csl = pl.ds(0, CHUNK)
            pltpu.sync_copy(src.at[pl.ds(off, CHUNK)], idx_s[0].at[csl])
            pltpu.sync_copy(dst.at[pl.ds(off, CHUNK)], idx_d[0].at[csl])
            pltpu.sync_copy(src_buf.at[idx_s[0].at[csl]], rows0.at[csl])
            pltpu.sync_copy(rows0.at[csl], acc.at[idx_d[0].at[csl]], add=True)

        fire_idx(0, 0)
        fire_idx(1, 1)
        fire_idx(2, 2)
        launch_g(0, 0)

        @pl.loop(0, (N_BLKS - 3) // 4)
        def _(t):
            b0 = 4 * t
            launch_g(1, 1)
            finish(0, 0)
            fire_idx(3, b0 + 3)
            launch_g(0, 2)
            finish(1, 1)
            fire_idx(0, b0 + 4)
            launch_g(1, 3)
            finish(0, 2)
            fire_idx(1, b0 + 5)
            launch_g(0, 0)
            finish(1, 3)
            fire_idx(2, b0 + 6)

        launch_g(1, 1)
        finish(0, 0)
        launch_g(0, 2)
        finish(1, 1)
        finish(0, 2)
        plsc.subcore_barrier()

        for k in range(N_WCHUNKS):
            r0 = rbase + k * WCHUNK
            pltpu.sync_copy(acc.at[pl.ds(r0, WCHUNK)], wbuf)
            if relu:
                @pl.loop(0, WCHUNK)
                def _(r):
                    for c in range(HALF // LANES):
                        v = wbuf[r, pl.ds(c * LANES, LANES)]
                        wbuf[r, pl.ds(c * LANES, LANES)] = jnp.maximum(v, 0.0)
            pltpu.sync_copy(wbuf, dst_buf.at[pl.ds(r0, WCHUNK)])
        plsc.subcore_barrier()

    @pl.when(cid == 0)
    def _():
        layer(x_lo, h1_lo, True)
        layer(h1_lo, h2_lo, False)
        layer(h2_lo, o_lo, False)

    @pl.when(cid == 1)
    def _():
        layer(x_hi, h1_hi, True)
        layer(h1_hi, h2_hi, False)
        layer(h2_hi, o_hi, False)


def kernel(x, edge_index):
    src = edge_index[0].astype(jnp.int32)
    dst = edge_index[1].astype(jnp.int32)
    x_lo = x[:, :HALF]
    x_hi = x[:, HALF:]
    *_, o_lo, o_hi = _conv3(x_lo, x_hi, src, dst)
    return jnp.concatenate([o_lo[:N_NODES], o_hi[:N_NODES]], axis=1)


# speedup vs baseline: 15.3806x; 1.0139x over previous
import functools

import jax
import jax.numpy as jnp
from jax import lax
from jax.experimental import pallas as pl
from jax.experimental.pallas import tpu as pltpu
from jax.experimental.pallas import tpu_sc as plsc

N_NODES = 10000
D_FEAT = 128
HALF = D_FEAT // 2
N_EDGES = 320000

N_TILES = 16
CHUNK = 128
N_CHUNKS = N_EDGES // CHUNK
CHUNKS_PER_TILE = N_CHUNKS // N_TILES
EDGES_PER_TILE = CHUNKS_PER_TILE * CHUNK
BLK = 4
BLK_E = BLK * CHUNK
N_BLKS = CHUNKS_PER_TILE // BLK
EXTRA_TILES = N_CHUNKS - N_TILES * CHUNKS_PER_TILE
EXTRA_BASE = N_TILES * EDGES_PER_TILE
N_PAD = 10240
ROWS_PER_TILE = N_PAD // N_TILES
WCHUNK = 128
N_WCHUNKS = ROWS_PER_TILE // WCHUNK
LANES = 16

_mesh = plsc.VectorSubcoreMesh(
    core_axis_name="c", subcore_axis_name="s", num_cores=2
)

_half = jax.ShapeDtypeStruct((N_PAD, HALF), jnp.float32)


@functools.partial(
    pl.kernel,
    out_type=(_half,) * 6,
    mesh=_mesh,
    compiler_params=pltpu.CompilerParams(use_tc_tiling_on_sc=False),
    scratch_types=[
        pltpu.VMEM_SHARED((N_PAD, HALF), jnp.float32),
        pltpu.VMEM((BLK_E, HALF), jnp.float32),
        pltpu.VMEM((BLK_E, HALF), jnp.float32),
        [pltpu.VMEM((BLK_E,), jnp.int32)] * 4,
        [pltpu.VMEM((BLK_E,), jnp.int32)] * 4,
        [pltpu.SemaphoreType.DMA] * 4,
        pltpu.VMEM((WCHUNK, HALF), jnp.float32),
        pltpu.VMEM((WCHUNK, HALF), jnp.float32),
        pltpu.SemaphoreType.DMA,
        pltpu.SemaphoreType.DMA,
        pltpu.SemaphoreType.DMA,
        pltpu.SemaphoreType.DMA,
    ],
)
def _conv3(x_lo, x_hi, src, dst,
           h1_lo, h1_hi, h2_lo, h2_hi, o_lo, o_hi,
           acc, rows0, rows1, idx_s, idx_d, isem,
           wbuf, zbuf, gsem0, gsem1, ssem0, ssem1):
    cid = lax.axis_index("c")
    wid = lax.axis_index("s")
    ebase = wid * EDGES_PER_TILE
    rbase = wid * ROWS_PER_TILE

    zeros = jnp.zeros((LANES,), jnp.float32)

    def layer(src_buf, dst_buf, relu):

        rbufs = ((rows0, gsem0, ssem0), (rows1, gsem1, ssem1))

        def fire_idx(s, blk):
            off = ebase + blk * BLK_E
            pltpu.async_copy(src.at[pl.ds(off, BLK_E)], idx_s[s], isem[s])
            pltpu.async_copy(dst.at[pl.ds(off, BLK_E)], idx_d[s], isem[s])

        def wait_idx(s):
            pltpu.make_async_copy(src.at[pl.ds(0, BLK_E)], idx_s[s], isem[s]).wait()
            pltpu.make_async_copy(dst.at[pl.ds(0, BLK_E)], idx_d[s], isem[s]).wait()

        def launch_g(b, s):
            r_ref, gsem, _ = rbufs[b]
            wait_idx(s)
            for k in range(BLK):
                sl = pl.ds(k * CHUNK, CHUNK)
                pltpu.async_copy(src_buf.at[idx_s[s].at[sl]], r_ref.at[sl], gsem)

        def finish(b, s):
            r_ref, gsem, ssem = rbufs[b]
            descs = []
            for k in range(BLK):
                sl = pl.ds(k * CHUNK, CHUNK)
                pltpu.make_async_copy(
                    src_buf.at[idx_s[s].at[sl]], r_ref.at[sl], gsem).wait()
                descs.append(pltpu.async_copy(
                    r_ref.at[sl], acc.at[idx_d[s].at[sl]], ssem, add=True))
            for d in descs:
                d.wait()

        @pl.when(wid < EXTRA_TILES)
        def _():
            off = EXTRA_BASE + wid * CHUNK
            csl = pl.ds(0, CHUNK)
            pltpu.sync_copy(src.at[pl.ds(off, CHUNK)], idx_s[0].at[csl])
            pltpu.sync_copy(dst.at[pl.ds(off, CHUNK)], idx_d[0].at[csl])
            pltpu.sync_copy(src_buf.at[idx_s[0].at[csl]], rows0.at[csl])
            pltpu.sync_copy(rows0.at[csl], acc.at[idx_d[0].at[csl]], add=True)

        fire_idx(0, 0)
        fire_idx(1, 1)
        fire_idx(2, 2)
        launch_g(0, 0)

        @pl.loop(0, (N_BLKS - 3) // 4)
        def _(t):
            b0 = 4 * t
            launch_g(1, 1)
            finish(0, 0)
            fire_idx(3, b0 + 3)
            launch_g(0, 2)
            finish(1, 1)
            fire_idx(0, b0 + 4)
            launch_g(1, 3)
            finish(0, 2)
            fire_idx(1, b0 + 5)
            launch_g(0, 0)
            finish(1, 3)
            fire_idx(2, b0 + 6)

        launch_g(1, 1)
        finish(0, 0)
        launch_g(0, 2)
        finish(1, 1)
        finish(0, 2)
        plsc.subcore_barrier()

        zdescs = []
        for k in range(N_WCHUNKS):
            r0 = rbase + k * WCHUNK
            pltpu.sync_copy(acc.at[pl.ds(r0, WCHUNK)], wbuf)
            zdescs.append(
                pltpu.async_copy(zbuf, acc.at[pl.ds(r0, WCHUNK)], ssem0))
            if relu:
                @pl.loop(0, WCHUNK)
                def _(r):
                    for c in range(HALF // LANES):
                        v = wbuf[r, pl.ds(c * LANES, LANES)]
                        wbuf[r, pl.ds(c * LANES, LANES)] = jnp.maximum(v, 0.0)
            pltpu.sync_copy(wbuf, dst_buf.at[pl.ds(r0, WCHUNK)])
        for d in zdescs:
            d.wait()
        plsc.subcore_barrier()

    @pl.loop(0, WCHUNK)
    def _(r):
        for c in range(HALF // LANES):
            zbuf[r, pl.ds(c * LANES, LANES)] = zeros

    for k in range(N_WCHUNKS):
        pltpu.sync_copy(zbuf, acc.at[pl.ds(rbase + k * WCHUNK, WCHUNK)])
    plsc.subcore_barrier()

    @pl.when(cid == 0)
    def _():
        layer(x_lo, h1_lo, True)
        layer(h1_lo, h2_lo, False)
        layer(h2_lo, o_lo, False)

    @pl.when(cid == 1)
    def _():
        layer(x_hi, h1_hi, True)
        layer(h1_hi, h2_hi, False)
        layer(h2_hi, o_hi, False)


def kernel(x, edge_index):
    src = edge_index[0].astype(jnp.int32)
    dst = edge_index[1].astype(jnp.int32)
    x_lo = x[:, :HALF]
    x_hi = x[:, HALF:]
    *_, o_lo, o_hi = _conv3(x_lo, x_hi, src, dst)
    return jnp.concatenate([o_lo[:N_NODES], o_hi[:N_NODES]], axis=1)
